# baseline (speedup 1.0000x reference)
import functools

import jax
import jax.numpy as jnp
from jax import lax
from jax.experimental import pallas as pl
from jax.experimental.pallas import tpu as pltpu
from jax.experimental.pallas import tpu_sc as plsc

NUM_CLASSES = 100000
FEAT_DIM = 128
BATCH = 4096

_NC = 2
_NS = 16
_NW = _NC * _NS
_L = 16
_BPW = BATCH // _NW
_CHUNKS = FEAT_DIM // _L
_CHUNK_ROWS = (16, 48, 64)
_GCH = len(_CHUNK_ROWS)


def _sc_partials(label, feat, centers):
    mesh = plsc.VectorSubcoreMesh(core_axis_name="c", subcore_axis_name="s")

    @functools.partial(
        pl.kernel,
        out_type=jax.ShapeDtypeStruct((_NW // 8, 8 * _L), jnp.float32),
        mesh=mesh,
        scratch_types=[
            pltpu.VMEM((_BPW,), jnp.int32),
            pltpu.VMEM((_BPW, FEAT_DIM), jnp.float32),
            pltpu.VMEM((_BPW, FEAT_DIM), jnp.float32),
            pltpu.VMEM((_L,), jnp.float32),
        ] + [pltpu.SemaphoreType.DMA] * (3 * _GCH),
    )
    def k(label_hbm, feat_hbm, centers_hbm, out_hbm, idx_v, rows_v, feat_v,
          acc_v, *sems):
        wid = lax.axis_index("s") * _NC + lax.axis_index("c")
        base = wid * _BPW

        offs = [sum(_CHUNK_ROWS[:g]) for g in range(_GCH)]
        lcopies, fcopies = [], []
        for g in range(_GCH):
            sl = pl.ds(offs[g], _CHUNK_ROWS[g])
            lcopies.append(pltpu.async_copy(
                label_hbm.at[pl.ds(base + offs[g], _CHUNK_ROWS[g])],
                idx_v.at[sl], sems[3 * g]))
            fcopies.append(pltpu.async_copy(
                feat_hbm.at[pl.ds(base + offs[g], _CHUNK_ROWS[g]), :],
                feat_v.at[sl, :], sems[3 * g + 1]))
        gathers = []
        for g in range(_GCH):
            sl = pl.ds(offs[g], _CHUNK_ROWS[g])
            lcopies[g].wait()
            gathers.append(pltpu.async_copy(
                centers_hbm.at[idx_v.at[sl]],
                rows_v.at[sl, :], sems[3 * g + 2]))

        zero = jnp.zeros((_L,), jnp.float32)

        def row_body(r, accs):
            out = []
            for c in range(_CHUNKS):
                d = feat_v[r, pl.ds(c * _L, _L)] - rows_v[r, pl.ds(c * _L, _L)]
                out.append(accs[c] + d * d)
            return tuple(out)

        accs = (zero,) * _CHUNKS
        for g in range(_GCH):
            gathers[g].wait()
            fcopies[g].wait()
            accs = plsc.parallel_loop(
                offs[g], offs[g] + _CHUNK_ROWS[g], unroll=4,
                carry=accs)(row_body)

        acc = accs[0]
        for c in range(1, _CHUNKS):
            acc = acc + accs[c]
        acc_v[...] = acc
        pltpu.sync_copy(acc_v, out_hbm.at[wid // 8, pl.ds((wid % 8) * _L, _L)])

    return k(label, feat, centers)


def _tc_reduce(partials):
    def red(x_ref, o_ref):
        o_ref[...] = (jnp.sum(x_ref[...]) * 0.5).reshape(1, 1)

    return pl.pallas_call(
        red,
        out_shape=jax.ShapeDtypeStruct((1, 1), jnp.float32),
    )(partials)


@jax.jit
def kernel(label, feat, centers):
    label = label.astype(jnp.int32)
    partials = _sc_partials(label, feat, centers)
    return _tc_reduce(partials).reshape(())


# speedup vs baseline: 1.0216x; 1.0216x over previous
import functools

import jax
import jax.numpy as jnp
from jax import lax
from jax.experimental import pallas as pl
from jax.experimental.pallas import tpu as pltpu
from jax.experimental.pallas import tpu_sc as plsc

NUM_CLASSES = 100000
FEAT_DIM = 128
BATCH = 4096

_NC = 2
_NS = 16
_NW = _NC * _NS
_L = 16
_BPW = BATCH // _NW
_CHUNKS = FEAT_DIM // _L
_CHUNK_ROWS = (32, 96)
_GCH = len(_CHUNK_ROWS)


def _sc_partials(label, feat, centers):
    mesh = plsc.VectorSubcoreMesh(core_axis_name="c", subcore_axis_name="s")

    @functools.partial(
        pl.kernel,
        out_type=jax.ShapeDtypeStruct((_NW // 8, 8 * _L), jnp.float32),
        mesh=mesh,
        scratch_types=[
            pltpu.VMEM((_BPW,), jnp.int32),
            pltpu.VMEM((_BPW, FEAT_DIM), jnp.float32),
            pltpu.VMEM((_BPW, FEAT_DIM), jnp.float32),
            pltpu.VMEM((_L,), jnp.float32),
        ] + [pltpu.SemaphoreType.DMA] * (3 * _GCH),
    )
    def k(label_hbm, feat_hbm, centers_hbm, out_hbm, idx_v, rows_v, feat_v,
          acc_v, *sems):
        wid = lax.axis_index("s") * _NC + lax.axis_index("c")
        base = wid * _BPW

        offs = [sum(_CHUNK_ROWS[:g]) for g in range(_GCH)]
        lcopies, fcopies = [], []
        for g in range(_GCH):
            sl = pl.ds(offs[g], _CHUNK_ROWS[g])
            lcopies.append(pltpu.async_copy(
                label_hbm.at[pl.ds(base + offs[g], _CHUNK_ROWS[g])],
                idx_v.at[sl], sems[3 * g]))
            fcopies.append(pltpu.async_copy(
                feat_hbm.at[pl.ds(base + offs[g], _CHUNK_ROWS[g]), :],
                feat_v.at[sl, :], sems[3 * g + 1]))
        gathers = []
        for g in range(_GCH):
            sl = pl.ds(offs[g], _CHUNK_ROWS[g])
            lcopies[g].wait()
            gathers.append(pltpu.async_copy(
                centers_hbm.at[idx_v.at[sl]],
                rows_v.at[sl, :], sems[3 * g + 2]))

        zero = jnp.zeros((_L,), jnp.float32)

        def row_body(r, accs):
            out = []
            for c in range(_CHUNKS):
                d = feat_v[r, pl.ds(c * _L, _L)] - rows_v[r, pl.ds(c * _L, _L)]
                out.append(accs[c] + d * d)
            return tuple(out)

        accs = (zero,) * _CHUNKS
        for g in range(_GCH):
            gathers[g].wait()
            fcopies[g].wait()
            accs = plsc.parallel_loop(
                offs[g], offs[g] + _CHUNK_ROWS[g], unroll=4,
                carry=accs)(row_body)

        acc = accs[0]
        for c in range(1, _CHUNKS):
            acc = acc + accs[c]
        acc_v[...] = acc
        pltpu.sync_copy(acc_v, out_hbm.at[wid // 8, pl.ds((wid % 8) * _L, _L)])

    return k(label, feat, centers)


def _tc_reduce(partials):
    def red(x_ref, o_ref):
        o_ref[...] = (jnp.sum(x_ref[...]) * 0.5).reshape(1, 1)

    return pl.pallas_call(
        red,
        out_shape=jax.ShapeDtypeStruct((1, 1), jnp.float32),
    )(partials)


@jax.jit
def kernel(label, feat, centers):
    label = label.astype(jnp.int32)
    partials = _sc_partials(label, feat, centers)
    return _tc_reduce(partials).reshape(())
